# baseline (speedup 1.0000x reference)
import functools

import jax
import jax.numpy as jnp
from jax import lax
from jax.experimental import pallas as pl
from jax.experimental.pallas import tpu as pltpu
from jax.experimental.pallas import tpu_sc as plsc

D_MODEL = 768
LANES = 16
COLS = D_MODEL // LANES
ROWS_PER_ITER = 4


def _sc_embed(idx_3d, tok_table, pos_table, batch, seq_len):
    n_tok = batch * seq_len
    info = plsc.get_sparse_core_info()
    nc, ns = info.num_cores, info.num_subcores
    nw = nc * ns
    pos_per_w = seq_len // nw
    ch = 32
    hsub = pos_per_w // ch
    n_ch = batch * hsub

    mesh = plsc.VectorSubcoreMesh(core_axis_name="c", subcore_axis_name="s")

    @functools.partial(
        pl.kernel,
        mesh=mesh,
        out_type=jax.ShapeDtypeStruct((n_tok, D_MODEL), jnp.float32),
        scratch_types=[
            pltpu.VMEM((batch, pos_per_w), jnp.int32),
            pltpu.VMEM((pos_per_w, D_MODEL), jnp.float32),
            pltpu.VMEM((ch, D_MODEL), jnp.float32),
            pltpu.VMEM((ch, D_MODEL), jnp.float32),
            pltpu.SemaphoreType.DMA,
            pltpu.SemaphoreType.DMA,
            pltpu.SemaphoreType.DMA,
            pltpu.SemaphoreType.DMA,
            pltpu.SemaphoreType.DMA,
        ],
    )
    def k(idx_hbm, tok_hbm, pos_hbm, out_hbm,
          idx_v, pos_v, buf_a, buf_b, g0, g1, o0, o1, psem):
        wid = lax.axis_index("s") * nc + lax.axis_index("c")
        pltpu.sync_copy(idx_hbm.at[0, wid], idx_v.at[0])
        gather0 = pltpu.async_copy(
            tok_hbm.at[idx_v.at[0, pl.ds(0, ch)]], buf_a, g0)
        pos_cp = pltpu.async_copy(
            pos_hbm.at[pl.ds(wid * pos_per_w, pos_per_w)], pos_v, psem)
        for b in range(1, batch):
            pltpu.sync_copy(idx_hbm.at[b, wid], idx_v.at[b])
        pos_cp.wait()

        bufs = (buf_a, buf_b)
        gsems = (g0, g1)
        osems = (o0, o1)

        out_copies = [None, None]
        cur_gather = gather0
        for r in range(n_ch):
            cur = r & 1
            nxt = 1 - cur
            b, h = divmod(r, hsub)
            if r + 1 < n_ch:
                nb, nh = divmod(r + 1, hsub)
                if out_copies[nxt] is not None:
                    out_copies[nxt].wait()
                next_gather = pltpu.async_copy(
                    tok_hbm.at[idx_v.at[nb, pl.ds(nh * ch, ch)]],
                    bufs[nxt], gsems[nxt])
            cur_gather.wait()
            buf = bufs[cur]

            def add_body(it, carry, _h=h, _buf=buf):
                row = it * ROWS_PER_ITER
                for rr in range(ROWS_PER_ITER):
                    for cc in range(COLS):
                        sl = pl.ds(cc * LANES, LANES)
                        x = pos_v[_h * ch + row + rr, sl]
                        plsc.addupdate(_buf.at[row + rr, sl], x)
                return carry

            lax.fori_loop(0, ch // ROWS_PER_ITER, add_body, 0)
            out_base = b * seq_len + wid * pos_per_w + h * ch
            out_copies[cur] = pltpu.async_copy(
                buf, out_hbm.at[pl.ds(out_base, ch)], osems[cur])
            if r + 1 < n_ch:
                cur_gather = next_gather
        out_copies[0].wait()
        out_copies[1].wait()

    return k(idx_3d, tok_table, pos_table)


def kernel(inputs, tok_table, pos_table):
    b, l = inputs.shape
    nw = 32
    idx_3d = inputs.reshape(b, nw, l // nw)
    out = _sc_embed(idx_3d, tok_table, pos_table, b, l)
    return out.reshape(b, l, D_MODEL)


# speedup vs baseline: 1.1392x; 1.1392x over previous
import functools

import jax
import jax.numpy as jnp
from jax import lax
from jax.experimental import pallas as pl
from jax.experimental.pallas import tpu as pltpu
from jax.experimental.pallas import tpu_sc as plsc

D_MODEL = 768
LANES = 16
COLS = D_MODEL // LANES
ROWS_PER_ITER = 4


def _sc_embed(idx_3d, tok_table, pos_table, batch, seq_len):
    n_tok = batch * seq_len
    info = plsc.get_sparse_core_info()
    nc, ns = info.num_cores, info.num_subcores
    nw = nc * ns
    pos_per_w = seq_len // nw
    ch = 32
    hsub = pos_per_w // ch
    n_ch = batch * hsub

    mesh = plsc.VectorSubcoreMesh(core_axis_name="c", subcore_axis_name="s")

    @functools.partial(
        pl.kernel,
        mesh=mesh,
        out_type=jax.ShapeDtypeStruct((n_tok, D_MODEL), jnp.float32),
        scratch_types=[
            pltpu.VMEM((batch, pos_per_w), jnp.int32),
            pltpu.VMEM((pos_per_w, D_MODEL), jnp.float32),
            pltpu.VMEM((ch, D_MODEL), jnp.float32),
            pltpu.VMEM((ch, D_MODEL), jnp.float32),
            pltpu.SemaphoreType.DMA,
            pltpu.SemaphoreType.DMA,
            pltpu.SemaphoreType.DMA,
            pltpu.SemaphoreType.DMA,
            pltpu.SemaphoreType.DMA,
        ],
    )
    def k(idx_hbm, tok_hbm, pos_hbm, out_hbm,
          idx_v, pos_v, buf_a, buf_b, g0, g1, o0, o1, psem):
        wid = lax.axis_index("s") * nc + lax.axis_index("c")
        pltpu.sync_copy(idx_hbm.at[0, wid], idx_v.at[0])
        gather0 = pltpu.async_copy(
            tok_hbm.at[idx_v.at[0, pl.ds(0, ch)]], buf_a, g0)
        pos_cp = pltpu.async_copy(
            pos_hbm.at[pl.ds(wid * pos_per_w, pos_per_w)], pos_v, psem)
        for b in range(1, batch):
            pltpu.sync_copy(idx_hbm.at[b, wid], idx_v.at[b])
        pos_cp.wait()

        bufs = (buf_a, buf_b)
        gsems = (g0, g1)
        osems = (o0, o1)

        out_copies = [None, None]
        cur_gather = gather0
        for r in range(n_ch):
            cur = r & 1
            nxt = 1 - cur
            b, h = divmod(r, hsub)
            if r + 1 < n_ch:
                nb, nh = divmod(r + 1, hsub)
                if out_copies[nxt] is not None:
                    out_copies[nxt].wait()
                next_gather = pltpu.async_copy(
                    tok_hbm.at[idx_v.at[nb, pl.ds(nh * ch, ch)]],
                    bufs[nxt], gsems[nxt])
            cur_gather.wait()
            buf = bufs[cur]

            @plsc.parallel_loop(0, ch, step=1, unroll=ROWS_PER_ITER)
            def add_body(row, _h=h, _buf=buf):
                for cc in range(COLS):
                    sl = pl.ds(cc * LANES, LANES)
                    x = pos_v[_h * ch + row, sl]
                    plsc.addupdate(_buf.at[row, sl], x)
            out_base = b * seq_len + wid * pos_per_w + h * ch
            out_copies[cur] = pltpu.async_copy(
                buf, out_hbm.at[pl.ds(out_base, ch)], osems[cur])
            if r + 1 < n_ch:
                cur_gather = next_gather
        out_copies[0].wait()
        out_copies[1].wait()

    return k(idx_3d, tok_table, pos_table)


def kernel(inputs, tok_table, pos_table):
    b, l = inputs.shape
    nw = 32
    idx_3d = inputs.reshape(b, nw, l // nw)
    out = _sc_embed(idx_3d, tok_table, pos_table, b, l)
    return out.reshape(b, l, D_MODEL)


# speedup vs baseline: 1.1888x; 1.0435x over previous
import functools

import jax
import jax.numpy as jnp
from jax import lax
from jax.experimental import pallas as pl
from jax.experimental.pallas import tpu as pltpu
from jax.experimental.pallas import tpu_sc as plsc

D_MODEL = 768
LANES = 16
COLS = D_MODEL // LANES
ROWS_PER_ITER = 8


def _sc_embed(idx_3d, tok_table, pos_table, batch, seq_len):
    n_tok = batch * seq_len
    info = plsc.get_sparse_core_info()
    nc, ns = info.num_cores, info.num_subcores
    nw = nc * ns
    pos_per_w = seq_len // nw
    ch = 32
    hsub = pos_per_w // ch
    n_ch = batch * hsub

    mesh = plsc.VectorSubcoreMesh(core_axis_name="c", subcore_axis_name="s")

    @functools.partial(
        pl.kernel,
        mesh=mesh,
        out_type=jax.ShapeDtypeStruct((n_tok, D_MODEL), jnp.float32),
        scratch_types=[
            pltpu.VMEM((batch, pos_per_w), jnp.int32),
            pltpu.VMEM((pos_per_w, D_MODEL), jnp.float32),
            pltpu.VMEM((ch, D_MODEL), jnp.float32),
            pltpu.VMEM((ch, D_MODEL), jnp.float32),
            pltpu.VMEM((ch, D_MODEL), jnp.float32),
            pltpu.SemaphoreType.DMA,
            pltpu.SemaphoreType.DMA,
            pltpu.SemaphoreType.DMA,
            pltpu.SemaphoreType.DMA,
            pltpu.SemaphoreType.DMA,
            pltpu.SemaphoreType.DMA,
            pltpu.SemaphoreType.DMA,
        ],
    )
    def k(idx_hbm, tok_hbm, pos_hbm, out_hbm,
          idx_v, pos_v, buf_a, buf_b, buf_c,
          g0, g1, g2, o0, o1, o2, psem):
        wid = lax.axis_index("s") * nc + lax.axis_index("c")
        pltpu.sync_copy(idx_hbm.at[0, wid], idx_v.at[0])
        gather0 = pltpu.async_copy(
            tok_hbm.at[idx_v.at[0, pl.ds(0, ch)]], buf_a, g0)
        pos_cp = pltpu.async_copy(
            pos_hbm.at[pl.ds(wid * pos_per_w, pos_per_w)], pos_v, psem)
        for b in range(1, batch):
            pltpu.sync_copy(idx_hbm.at[b, wid], idx_v.at[b])
        pos_cp.wait()

        nbuf = 3
        bufs = (buf_a, buf_b, buf_c)
        gsems = (g0, g1, g2)
        osems = (o0, o1, o2)

        out_copies = [None] * nbuf
        gathers = [None] * nbuf
        gathers[0] = gather0
        gathers[1] = pltpu.async_copy(
            tok_hbm.at[idx_v.at[0, pl.ds(ch, ch)]], bufs[1], gsems[1])
        for r in range(n_ch):
            cur = r % nbuf
            b, h = divmod(r, hsub)
            if r + 2 < n_ch:
                nxt = (r + 2) % nbuf
                nb, nh = divmod(r + 2, hsub)
                if out_copies[nxt] is not None:
                    out_copies[nxt].wait()
                gathers[nxt] = pltpu.async_copy(
                    tok_hbm.at[idx_v.at[nb, pl.ds(nh * ch, ch)]],
                    bufs[nxt], gsems[nxt])
            gathers[cur].wait()
            buf = bufs[cur]

            @plsc.parallel_loop(0, ch, step=1, unroll=ROWS_PER_ITER)
            def add_body(row, _h=h, _buf=buf):
                for cc in range(COLS):
                    sl = pl.ds(cc * LANES, LANES)
                    x = pos_v[_h * ch + row, sl]
                    plsc.addupdate(_buf.at[row, sl], x)
            out_base = b * seq_len + wid * pos_per_w + h * ch
            out_copies[cur] = pltpu.async_copy(
                buf, out_hbm.at[pl.ds(out_base, ch)], osems[cur])
        for oc in out_copies:
            oc.wait()

    return k(idx_3d, tok_table, pos_table)


def kernel(inputs, tok_table, pos_table):
    b, l = inputs.shape
    nw = 32
    idx_3d = inputs.reshape(b, nw, l // nw)
    out = _sc_embed(idx_3d, tok_table, pos_table, b, l)
    return out.reshape(b, l, D_MODEL)


# speedup vs baseline: 1.1895x; 1.0007x over previous
import functools

import jax
import jax.numpy as jnp
from jax import lax
from jax.experimental import pallas as pl
from jax.experimental.pallas import tpu as pltpu
from jax.experimental.pallas import tpu_sc as plsc

D_MODEL = 768
LANES = 16
COLS = D_MODEL // LANES
ROWS_PER_ITER = 8


def _sc_embed(idx_3d, tok_table, pos_table, batch, seq_len):
    n_tok = batch * seq_len
    info = plsc.get_sparse_core_info()
    nc, ns = info.num_cores, info.num_subcores
    nw = nc * ns
    pos_per_w = seq_len // nw
    ch = 32
    hsub = pos_per_w // ch
    n_ch = batch * hsub

    mesh = plsc.VectorSubcoreMesh(core_axis_name="c", subcore_axis_name="s")

    @functools.partial(
        pl.kernel,
        mesh=mesh,
        out_type=jax.ShapeDtypeStruct((n_tok, D_MODEL), jnp.float32),
        scratch_types=[
            pltpu.VMEM((batch, pos_per_w), jnp.int32),
            pltpu.VMEM((pos_per_w, D_MODEL), jnp.float32),
            pltpu.VMEM((ch, D_MODEL), jnp.float32),
            pltpu.VMEM((ch, D_MODEL), jnp.float32),
            pltpu.VMEM((ch, D_MODEL), jnp.float32),
            pltpu.SemaphoreType.DMA,
            pltpu.SemaphoreType.DMA,
            pltpu.SemaphoreType.DMA,
            pltpu.SemaphoreType.DMA,
            pltpu.SemaphoreType.DMA,
            pltpu.SemaphoreType.DMA,
            pltpu.SemaphoreType.DMA,
        ],
    )
    def k(idx_hbm, tok_hbm, pos_hbm, out_hbm,
          idx_v, pos_v, buf_a, buf_b, buf_c,
          g0, g1, g2, o0, o1, o2, psem):
        wid = lax.axis_index("s") * nc + lax.axis_index("c")
        pltpu.sync_copy(idx_hbm.at[wid], idx_v)
        gather0 = pltpu.async_copy(
            tok_hbm.at[idx_v.at[0, pl.ds(0, ch)]], buf_a, g0)
        pos_cp = pltpu.async_copy(
            pos_hbm.at[pl.ds(wid * pos_per_w, pos_per_w)], pos_v, psem)
        pos_cp.wait()

        nbuf = 3
        bufs = (buf_a, buf_b, buf_c)
        gsems = (g0, g1, g2)
        osems = (o0, o1, o2)

        out_copies = [None] * nbuf
        gathers = [None] * nbuf
        gathers[0] = gather0
        gathers[1] = pltpu.async_copy(
            tok_hbm.at[idx_v.at[0, pl.ds(ch, ch)]], bufs[1], gsems[1])
        for r in range(n_ch):
            cur = r % nbuf
            b, h = divmod(r, hsub)
            if r + 2 < n_ch:
                nxt = (r + 2) % nbuf
                nb, nh = divmod(r + 2, hsub)
                if out_copies[nxt] is not None:
                    out_copies[nxt].wait()
                gathers[nxt] = pltpu.async_copy(
                    tok_hbm.at[idx_v.at[nb, pl.ds(nh * ch, ch)]],
                    bufs[nxt], gsems[nxt])
            gathers[cur].wait()
            buf = bufs[cur]

            @plsc.parallel_loop(0, ch, step=1, unroll=ROWS_PER_ITER)
            def add_body(row, _h=h, _buf=buf):
                for cc in range(COLS):
                    sl = pl.ds(cc * LANES, LANES)
                    x = pos_v[_h * ch + row, sl]
                    plsc.addupdate(_buf.at[row, sl], x)
            out_base = b * seq_len + wid * pos_per_w + h * ch
            out_copies[cur] = pltpu.async_copy(
                buf, out_hbm.at[pl.ds(out_base, ch)], osems[cur])
        for oc in out_copies:
            oc.wait()

    return k(idx_3d, tok_table, pos_table)


def kernel(inputs, tok_table, pos_table):
    b, l = inputs.shape
    nw = 32
    idx_3d = inputs.reshape(b, nw, l // nw).transpose(1, 0, 2)
    out = _sc_embed(idx_3d, tok_table, pos_table, b, l)
    return out.reshape(b, l, D_MODEL)


# speedup vs baseline: 1.3050x; 1.0970x over previous
import functools

import jax
import jax.numpy as jnp
from jax import lax
from jax.experimental import pallas as pl
from jax.experimental.pallas import tpu as pltpu
from jax.experimental.pallas import tpu_sc as plsc

D_MODEL = 768
LANES = 16
COLS = D_MODEL // LANES
GROUP_ROWS = 16
ROWS_PER_ITER = 2


def _sc_embed(idx_3d, tok_table, pos_table, batch, seq_len):
    n_tok = batch * seq_len
    info = plsc.get_sparse_core_info()
    nc, ns = info.num_cores, info.num_subcores
    nw = nc * ns
    pos_per_w = seq_len // nw
    n_grp = pos_per_w // GROUP_ROWS

    mesh = plsc.VectorSubcoreMesh(core_axis_name="c", subcore_axis_name="s")

    @functools.partial(
        pl.kernel,
        mesh=mesh,
        out_type=jax.ShapeDtypeStruct((n_tok, D_MODEL), jnp.float32),
        scratch_types=(
            [pltpu.VMEM((batch, pos_per_w), jnp.int32)]
            + [pltpu.VMEM((GROUP_ROWS, D_MODEL), jnp.float32)] * 2
            + [pltpu.VMEM((GROUP_ROWS, D_MODEL), jnp.float32)] * (2 * batch)
            + [pltpu.SemaphoreType.DMA] * (2 + 4 * batch)
        ),
    )
    def k(idx_hbm, tok_hbm, pos_hbm, out_hbm, idx_v, *rest):
        posb = rest[:2]
        bufs = (rest[2:2 + batch], rest[2 + batch:2 + 2 * batch])
        sems = rest[2 + 2 * batch:]
        pisems = sems[:2]
        gsems = (sems[2:2 + batch], sems[2 + batch:2 + 2 * batch])
        osems = (sems[2 + 2 * batch:2 + 3 * batch],
                 sems[2 + 3 * batch:2 + 4 * batch])
        wid = lax.axis_index("s") * nc + lax.axis_index("c")
        pltpu.sync_copy(idx_hbm.at[wid], idx_v)

        posinits = [None] * 2
        gathers = [[None] * batch, [None] * batch]
        outs = [[None] * batch, [None] * batch]

        def issue_group(h):
            g = h % 2
            posinits[g] = pltpu.async_copy(
                pos_hbm.at[pl.ds(wid * pos_per_w + h * GROUP_ROWS,
                                 GROUP_ROWS)],
                posb[g], pisems[g])
            for b in range(batch):
                gathers[g][b] = pltpu.async_copy(
                    tok_hbm.at[idx_v.at[b, pl.ds(h * GROUP_ROWS,
                                                 GROUP_ROWS)]],
                    bufs[g][b], gsems[g][b])

        issue_group(0)
        for h in range(n_grp):
            g = h % 2
            if h + 1 < n_grp:
                if h >= 1:
                    for b in range(batch):
                        outs[g ^ 1][b].wait()
                issue_group(h + 1)
            posinits[g].wait()
            for b in range(batch):
                gathers[g][b].wait()
            my_bufs = bufs[g]
            my_pos = posb[g]

            @plsc.parallel_loop(0, GROUP_ROWS, step=1, unroll=ROWS_PER_ITER)
            def add_body(row, _bufs=my_bufs, _pos=my_pos):
                for cc in range(COLS):
                    sl = pl.ds(cc * LANES, LANES)
                    x = _pos[row, sl]
                    for b in range(batch):
                        plsc.addupdate(_bufs[b].at[row, sl], x)

            for b in range(batch):
                outs[g][b] = pltpu.async_copy(
                    my_bufs[b],
                    out_hbm.at[pl.ds(b * seq_len + wid * pos_per_w
                                     + h * GROUP_ROWS, GROUP_ROWS)],
                    osems[g][b])
        for grp in outs:
            for oc in grp:
                if oc is not None:
                    oc.wait()

    return k(idx_3d, tok_table, pos_table)


def kernel(inputs, tok_table, pos_table):
    b, l = inputs.shape
    nw = 32
    idx_3d = inputs.reshape(b, nw, l // nw).transpose(1, 0, 2)
    out = _sc_embed(idx_3d, tok_table, pos_table, b, l)
    return out.reshape(b, l, D_MODEL)
